# initial kernel scaffold (unmeasured)
import jax
import jax.numpy as jnp
from jax import lax
from jax.experimental import pallas as pl
from jax.experimental.pallas import tpu as pltpu

N_DEV = 4
SQ = 256
SKV_LOC = 4096
HQ = 32
HG = 8
DH = 128
DM = 1024
BLK = 64
SCALE = 0.08838834764831843


def kernel(x, Wq, K_ext, V_ext, Wo):
    def body(x_ref, wq_ref, k_hbm, v_hbm, wo_ref, out_ref,
             q_full, bias_ref, k_buf, v_buf, o_part, ml_part,
             o_a2a, ml_a2a, ar_buf,
             k_sem, v_sem, q_send, q_recv, o_send, o_recv,
             ml_send, ml_recv, ar_send, ar_recv):
        p = lax.axis_index("i")

        bar = pltpu.get_barrier_semaphore()
        for d in range(1, N_DEV):
            pl.semaphore_signal(
                bar, inc=1,
                device_id=(lax.rem(p + d, N_DEV),),
                device_id_type=pl.DeviceIdType.MESH,
            )
        pl.semaphore_wait(bar, N_DEV - 1)

        def kv_copy(h, slot):
            kc = pltpu.make_async_copy(
                k_hbm.at[0, :, h, :], k_buf.at[slot], k_sem.at[slot])
            vc = pltpu.make_async_copy(
                v_hbm.at[0, :, h, :], v_buf.at[slot], v_sem.at[slot])
            return kc, vc

        kc0, vc0 = kv_copy(0, 0)
        kc0.start()
        vc0.start()

        xb = x_ref[0].astype(jnp.bfloat16)
        wqb = wq_ref[...].astype(jnp.bfloat16)
        q = jnp.dot(xb, wqb, preferred_element_type=jnp.float32)
        q_full[p, :, :] = q.astype(jnp.bfloat16)

        def exchange(src_buf, dst_buf, send_sems, recv_sems):
            sends = []
            for d in range(1, N_DEV):
                dst = lax.rem(p + d, N_DEV)
                r = pltpu.make_async_remote_copy(
                    src_ref=src_buf.at[p],
                    dst_ref=dst_buf.at[p],
                    send_sem=send_sems.at[dst],
                    recv_sem=recv_sems.at[p],
                    device_id=(dst,),
                    device_id_type=pl.DeviceIdType.MESH,
                )
                r.start()
                sends.append(r)
            return sends

        def wait_recvs(src_buf, dst_buf, send_sems, recv_sems):
            for d in range(1, N_DEV):
                src = lax.rem(p + d, N_DEV)
                pltpu.make_async_remote_copy(
                    src_ref=src_buf.at[src],
                    dst_ref=dst_buf.at[src],
                    send_sem=send_sems.at[src],
                    recv_sem=recv_sems.at[src],
                    device_id=(src,),
                    device_id_type=pl.DeviceIdType.MESH,
                ).wait_recv()

        q_sends = exchange(q_full, q_full, q_send, q_recv)

        ri = lax.broadcasted_iota(jnp.int32, (SQ, SKV_LOC), 0)
        ci = lax.broadcasted_iota(jnp.int32, (SQ, SKV_LOC), 1)
        qb = ri // BLK
        kbg = p * (SKV_LOC // BLK) + ci // BLK
        mask = (qb == kbg) | (kbg == 0) | (lax.rem(qb + kbg, 3) == 0)
        bias_ref[...] = jnp.where(mask, 0.0, -1e9).astype(jnp.float32)

        wait_recvs(q_full, q_full, q_send, q_recv)
        for r in q_sends:
            r.wait_send()

        def head_body(h, carry):
            slot = lax.rem(h, 2)
            nslot = lax.rem(h + 1, 2)

            @pl.when(h + 1 < HQ)
            def _():
                kc, vc = kv_copy(h + 1, nslot)
                kc.start()
                vc.start()

            kc, vc = kv_copy(h, slot)
            kc.wait()
            vc.wait()

            g = h // HG
            h8 = lax.rem(h, HG)
            qh = q_full[g, :, pl.ds(h8 * DH, DH)]
            kh = k_buf[slot].astype(jnp.bfloat16)
            s = lax.dot_general(
                qh, kh, (((1,), (1,)), ((), ())),
                preferred_element_type=jnp.float32)
            s = s * SCALE + bias_ref[...]
            m = jnp.max(s, axis=1)
            w = jnp.exp(s - m[:, None])
            l = jnp.sum(w, axis=1)
            vh = v_buf[slot].astype(jnp.bfloat16)
            o = jnp.dot(w.astype(jnp.bfloat16), vh,
                        preferred_element_type=jnp.float32)
            o_part[g, :, pl.ds(h8 * DH, DH)] = o.astype(jnp.bfloat16)
            ml_part[g, 0, h8, :] = m
            ml_part[g, 1, h8, :] = l
            return carry

        lax.fori_loop(0, HQ, head_body, 0)

        o_a2a[p, :, :] = o_part[p, :, :]
        ml_a2a[p, :, :, :] = ml_part[p, :, :, :]
        a2a_sends = []
        for d in range(1, N_DEV):
            dst = lax.rem(p + d, N_DEV)
            ro = pltpu.make_async_remote_copy(
                src_ref=o_part.at[dst], dst_ref=o_a2a.at[p],
                send_sem=o_send.at[dst], recv_sem=o_recv.at[p],
                device_id=(dst,), device_id_type=pl.DeviceIdType.MESH)
            rm = pltpu.make_async_remote_copy(
                src_ref=ml_part.at[dst], dst_ref=ml_a2a.at[p],
                send_sem=ml_send.at[dst], recv_sem=ml_recv.at[p],
                device_id=(dst,), device_id_type=pl.DeviceIdType.MESH)
            ro.start()
            rm.start()
            a2a_sends += [ro, rm]
        wait_recvs(o_part, o_a2a, o_send, o_recv)
        wait_recvs(ml_part, ml_a2a, ml_send, ml_recv)
        for r in a2a_sends:
            r.wait_send()

        mm = ml_a2a[:, 0, :, :]
        ll = ml_a2a[:, 1, :, :]
        mg = jnp.max(mm, axis=0)
        coef = jnp.exp(mm - mg[None, :, :])
        l_tot = jnp.sum(ll * coef, axis=0)
        ctx_cols = []
        for h8 in range(HG):
            oj = o_a2a[:, :, h8 * DH:(h8 + 1) * DH].astype(jnp.float32)
            c = coef[:, h8, :]
            o_tot = jnp.sum(oj * c[:, :, None], axis=0)
            ctx_cols.append(
                (o_tot / l_tot[h8][:, None]).astype(jnp.bfloat16))
        ctx = jnp.concatenate(ctx_cols, axis=1)

        wob = wo_ref[...].astype(jnp.bfloat16)
        out_p = jnp.dot(ctx, wob, preferred_element_type=jnp.float32)
        ar_buf[p, :, :] = out_p.astype(jnp.bfloat16)
        ar_sends = exchange(ar_buf, ar_buf, ar_send, ar_recv)
        wait_recvs(ar_buf, ar_buf, ar_send, ar_recv)
        for r in ar_sends:
            r.wait_send()
        out_ref[0, :, :] = jnp.sum(ar_buf[...].astype(jnp.float32), axis=0)

    f32 = jnp.float32
    bf16 = jnp.bfloat16
    return pl.pallas_call(
        body,
        out_shape=jax.ShapeDtypeStruct((1, SQ, DM), f32),
        in_specs=[
            pl.BlockSpec(memory_space=pltpu.VMEM),
            pl.BlockSpec(memory_space=pltpu.VMEM),
            pl.BlockSpec(memory_space=pltpu.ANY),
            pl.BlockSpec(memory_space=pltpu.ANY),
            pl.BlockSpec(memory_space=pltpu.VMEM),
        ],
        out_specs=pl.BlockSpec(memory_space=pltpu.VMEM),
        scratch_shapes=[
            pltpu.VMEM((N_DEV, SQ, DM), bf16),
            pltpu.VMEM((SQ, SKV_LOC), f32),
            pltpu.VMEM((2, SKV_LOC, DH), f32),
            pltpu.VMEM((2, SKV_LOC, DH), f32),
            pltpu.VMEM((N_DEV, SQ, DM), bf16),
            pltpu.VMEM((N_DEV, 2, HG, SQ), f32),
            pltpu.VMEM((N_DEV, SQ, DM), bf16),
            pltpu.VMEM((N_DEV, 2, HG, SQ), f32),
            pltpu.VMEM((N_DEV, SQ, DM), bf16),
            pltpu.SemaphoreType.DMA((2,)),
            pltpu.SemaphoreType.DMA((2,)),
            pltpu.SemaphoreType.DMA((N_DEV,)),
            pltpu.SemaphoreType.DMA((N_DEV,)),
            pltpu.SemaphoreType.DMA((N_DEV,)),
            pltpu.SemaphoreType.DMA((N_DEV,)),
            pltpu.SemaphoreType.DMA((N_DEV,)),
            pltpu.SemaphoreType.DMA((N_DEV,)),
            pltpu.SemaphoreType.DMA((N_DEV,)),
            pltpu.SemaphoreType.DMA((N_DEV,)),
        ],
        compiler_params=pltpu.CompilerParams(collective_id=0),
    )(x, Wq, K_ext, V_ext, Wo)


# baseline (device time: 129156 ns/iter reference)
import jax
import jax.numpy as jnp
from jax import lax
from jax.experimental import pallas as pl
from jax.experimental.pallas import tpu as pltpu

N_DEV = 4
SQ = 256
SKV_LOC = 4096
HQ = 32
HG = 8
DH = 128
DM = 1024
BLK = 64
SCALE = 0.08838834764831843


def kernel(x, Wq, K_ext, V_ext, Wo):
    def body(x_ref, wq_ref, k_hbm, v_hbm, wo_ref, out_ref,
             q_full, bias_ref, k_buf, v_buf, o_part, ml_part,
             o_a2a, ml_a2a, ar_buf,
             k_sem, v_sem, q_send, q_recv, o_send, o_recv,
             ml_send, ml_recv, ar_send, ar_recv):
        p = lax.axis_index("i")

        bar = pltpu.get_barrier_semaphore()
        for d in range(1, N_DEV):
            pl.semaphore_signal(
                bar, inc=1,
                device_id=(lax.rem(p + d, N_DEV),),
                device_id_type=pl.DeviceIdType.MESH,
            )
        pl.semaphore_wait(bar, N_DEV - 1)

        def kv_copy(h, slot):
            kc = pltpu.make_async_copy(
                k_hbm.at[0, :, h, :], k_buf.at[slot], k_sem.at[slot])
            vc = pltpu.make_async_copy(
                v_hbm.at[0, :, h, :], v_buf.at[slot], v_sem.at[slot])
            return kc, vc

        kc0, vc0 = kv_copy(0, 0)
        kc0.start()
        vc0.start()

        xb = x_ref[0].astype(jnp.bfloat16)
        wqb = wq_ref[...].astype(jnp.bfloat16)
        q = jnp.dot(xb, wqb, preferred_element_type=jnp.float32)
        q_full[p, :, :] = q.astype(jnp.bfloat16)

        def exchange(src_buf, dst_buf, send_sems, recv_sems):
            sends = []
            for d in range(1, N_DEV):
                dst = lax.rem(p + d, N_DEV)
                r = pltpu.make_async_remote_copy(
                    src_ref=src_buf.at[p],
                    dst_ref=dst_buf.at[p],
                    send_sem=send_sems.at[dst],
                    recv_sem=recv_sems.at[p],
                    device_id=(dst,),
                    device_id_type=pl.DeviceIdType.MESH,
                )
                r.start()
                sends.append(r)
            return sends

        def wait_recvs(src_buf, dst_buf, send_sems, recv_sems):
            for d in range(1, N_DEV):
                src = lax.rem(p + d, N_DEV)
                pltpu.make_async_remote_copy(
                    src_ref=src_buf.at[src],
                    dst_ref=dst_buf.at[src],
                    send_sem=send_sems.at[src],
                    recv_sem=recv_sems.at[src],
                    device_id=(src,),
                    device_id_type=pl.DeviceIdType.MESH,
                ).wait_recv()

        q_sends = exchange(q_full, q_full, q_send, q_recv)

        ri = lax.broadcasted_iota(jnp.int32, (SQ, SKV_LOC), 0)
        ci = lax.broadcasted_iota(jnp.int32, (SQ, SKV_LOC), 1)
        qb = ri // BLK
        kbg = p * (SKV_LOC // BLK) + ci // BLK
        mask = (qb == kbg) | (kbg == 0) | (lax.rem(qb + kbg, 3) == 0)
        bias_ref[...] = jnp.where(mask, 0.0, -1e9).astype(jnp.float32)

        wait_recvs(q_full, q_full, q_send, q_recv)
        for r in q_sends:
            r.wait_send()

        def head_body(h, carry):
            slot = lax.rem(h, 2)
            nslot = lax.rem(h + 1, 2)

            @pl.when(h + 1 < HQ)
            def _():
                kc, vc = kv_copy(h + 1, nslot)
                kc.start()
                vc.start()

            kc, vc = kv_copy(h, slot)
            kc.wait()
            vc.wait()

            g = h // HG
            h8 = lax.rem(h, HG)
            qh = q_full[g, :, pl.ds(h8 * DH, DH)]
            kh = k_buf[slot].astype(jnp.bfloat16)
            s = lax.dot_general(
                qh, kh, (((1,), (1,)), ((), ())),
                preferred_element_type=jnp.float32)
            s = s * SCALE + bias_ref[...]
            m = jnp.max(s, axis=1)
            w = jnp.exp(s - m[:, None])
            l = jnp.sum(w, axis=1)
            vh = v_buf[slot].astype(jnp.bfloat16)
            o = jnp.dot(w.astype(jnp.bfloat16), vh,
                        preferred_element_type=jnp.float32)
            o_part[g, :, pl.ds(h8 * DH, DH)] = o.astype(jnp.bfloat16)
            ml_part[g, 0, h8, :] = m
            ml_part[g, 1, h8, :] = l
            return carry

        lax.fori_loop(0, HQ, head_body, 0)

        o_a2a[p, :, :] = o_part[p, :, :]
        ml_a2a[p, :, :, :] = ml_part[p, :, :, :]
        a2a_sends = []
        for d in range(1, N_DEV):
            dst = lax.rem(p + d, N_DEV)
            ro = pltpu.make_async_remote_copy(
                src_ref=o_part.at[dst], dst_ref=o_a2a.at[p],
                send_sem=o_send.at[dst], recv_sem=o_recv.at[p],
                device_id=(dst,), device_id_type=pl.DeviceIdType.MESH)
            rm = pltpu.make_async_remote_copy(
                src_ref=ml_part.at[dst], dst_ref=ml_a2a.at[p],
                send_sem=ml_send.at[dst], recv_sem=ml_recv.at[p],
                device_id=(dst,), device_id_type=pl.DeviceIdType.MESH)
            ro.start()
            rm.start()
            a2a_sends += [ro, rm]
        wait_recvs(o_part, o_a2a, o_send, o_recv)
        wait_recvs(ml_part, ml_a2a, ml_send, ml_recv)
        for r in a2a_sends:
            r.wait_send()

        mm = ml_a2a[:, 0, :, :]
        ll = ml_a2a[:, 1, :, :]
        mg = jnp.max(mm, axis=0)
        coef = jnp.exp(mm - mg[None, :, :])
        l_tot = jnp.sum(ll * coef, axis=0)
        ctx_cols = []
        for h8 in range(HG):
            oj = o_a2a[:, :, h8 * DH:(h8 + 1) * DH].astype(jnp.float32)
            c = coef[:, h8, :]
            o_tot = jnp.sum(oj * c[:, :, None], axis=0)
            ctx_cols.append(
                (o_tot / l_tot[h8][:, None]).astype(jnp.bfloat16))
        ctx = jnp.concatenate(ctx_cols, axis=1)

        wob = wo_ref[...].astype(jnp.bfloat16)
        out_p = jnp.dot(ctx, wob, preferred_element_type=jnp.float32)
        ar_buf[p, :, :] = out_p.astype(jnp.bfloat16)
        ar_sends = exchange(ar_buf, ar_buf, ar_send, ar_recv)
        wait_recvs(ar_buf, ar_buf, ar_send, ar_recv)
        for r in ar_sends:
            r.wait_send()
        out_ref[0, :, :] = jnp.sum(ar_buf[...].astype(jnp.float32), axis=0)

    f32 = jnp.float32
    bf16 = jnp.bfloat16
    return pl.pallas_call(
        body,
        out_shape=jax.ShapeDtypeStruct((1, SQ, DM), f32),
        in_specs=[
            pl.BlockSpec(memory_space=pltpu.VMEM),
            pl.BlockSpec(memory_space=pltpu.VMEM),
            pl.BlockSpec(memory_space=pl.ANY),
            pl.BlockSpec(memory_space=pl.ANY),
            pl.BlockSpec(memory_space=pltpu.VMEM),
        ],
        out_specs=pl.BlockSpec(memory_space=pltpu.VMEM),
        scratch_shapes=[
            pltpu.VMEM((N_DEV, SQ, DM), bf16),
            pltpu.VMEM((SQ, SKV_LOC), f32),
            pltpu.VMEM((2, SKV_LOC, DH), f32),
            pltpu.VMEM((2, SKV_LOC, DH), f32),
            pltpu.VMEM((N_DEV, SQ, DM), bf16),
            pltpu.VMEM((N_DEV, 2, HG, SQ), f32),
            pltpu.VMEM((N_DEV, SQ, DM), bf16),
            pltpu.VMEM((N_DEV, 2, HG, SQ), f32),
            pltpu.VMEM((N_DEV, SQ, DM), bf16),
            pltpu.SemaphoreType.DMA((2,)),
            pltpu.SemaphoreType.DMA((2,)),
            pltpu.SemaphoreType.DMA((N_DEV,)),
            pltpu.SemaphoreType.DMA((N_DEV,)),
            pltpu.SemaphoreType.DMA((N_DEV,)),
            pltpu.SemaphoreType.DMA((N_DEV,)),
            pltpu.SemaphoreType.DMA((N_DEV,)),
            pltpu.SemaphoreType.DMA((N_DEV,)),
            pltpu.SemaphoreType.DMA((N_DEV,)),
            pltpu.SemaphoreType.DMA((N_DEV,)),
        ],
        compiler_params=pltpu.CompilerParams(collective_id=0),
    )(x, Wq, K_ext, V_ext, Wo)


# device time: 112127 ns/iter; 1.1519x vs baseline; 1.1519x over previous
import jax
import jax.numpy as jnp
from jax import lax
from jax.experimental import pallas as pl
from jax.experimental.pallas import tpu as pltpu

N_DEV = 4
SQ = 256
SKV_LOC = 4096
HQ = 32
HG = 8
DH = 128
DM = 1024
BLK = 64
SCALE = 0.08838834764831843


def kernel(x, Wq, K_ext, V_ext, Wo):
    def body(x_ref, wq_ref, k_hbm, v_hbm, wo_ref, out_ref,
             q_full, bias_ref, k_buf, v_buf, o_part, ml_part,
             o_a2a, ml_a2a, ar_buf,
             k_sem, v_sem, q_send, q_recv, o_send, o_recv,
             ml_send, ml_recv, ar_send, ar_recv):
        p = lax.axis_index("i")

        bar = pltpu.get_barrier_semaphore()
        for d in range(1, N_DEV):
            pl.semaphore_signal(
                bar, inc=1,
                device_id=(lax.rem(p + d, N_DEV),),
                device_id_type=pl.DeviceIdType.MESH,
            )
        pl.semaphore_wait(bar, N_DEV - 1)

        def kv_copy(h, slot):
            kc = pltpu.make_async_copy(
                k_hbm.at[0, :, h, :], k_buf.at[slot], k_sem.at[slot])
            vc = pltpu.make_async_copy(
                v_hbm.at[0, :, h, :], v_buf.at[slot], v_sem.at[slot])
            return kc, vc

        kc0, vc0 = kv_copy(p * HG, 0)
        kc0.start()
        vc0.start()

        xb = x_ref[0].astype(jnp.bfloat16)
        wqb = wq_ref[...].astype(jnp.bfloat16)
        q = jnp.dot(xb, wqb, preferred_element_type=jnp.float32)
        q_full[p, :, :] = q.astype(jnp.bfloat16)

        def exchange(src_buf, dst_buf, send_sems, recv_sems):
            sends = []
            for d in range(1, N_DEV):
                dst = lax.rem(p + d, N_DEV)
                r = pltpu.make_async_remote_copy(
                    src_ref=src_buf.at[p],
                    dst_ref=dst_buf.at[p],
                    send_sem=send_sems.at[dst],
                    recv_sem=recv_sems.at[p],
                    device_id=(dst,),
                    device_id_type=pl.DeviceIdType.MESH,
                )
                r.start()
                sends.append(r)
            return sends

        def wait_recvs(src_buf, dst_buf, send_sems, recv_sems):
            for d in range(1, N_DEV):
                src = lax.rem(p + d, N_DEV)
                pltpu.make_async_remote_copy(
                    src_ref=src_buf.at[src],
                    dst_ref=dst_buf.at[src],
                    send_sem=send_sems.at[src],
                    recv_sem=recv_sems.at[src],
                    device_id=(src,),
                    device_id_type=pl.DeviceIdType.MESH,
                ).wait_recv()

        q_sends = exchange(q_full, q_full, q_send, q_recv)

        ri = lax.broadcasted_iota(jnp.int32, (SQ, SKV_LOC), 0)
        ci = lax.broadcasted_iota(jnp.int32, (SQ, SKV_LOC), 1)
        qb = ri // BLK
        kbg = p * (SKV_LOC // BLK) + ci // BLK
        mask = (qb == kbg) | (kbg == 0) | (lax.rem(qb + kbg, 3) == 0)
        bias_ref[...] = jnp.where(mask, 0.0, -1e9).astype(jnp.float32)

        a2a_sends = []
        for d in range(N_DEV):
            g = lax.rem(p + d, N_DEV)

            if d > 0:
                pltpu.make_async_remote_copy(
                    src_ref=q_full.at[g], dst_ref=q_full.at[g],
                    send_sem=q_send.at[g], recv_sem=q_recv.at[g],
                    device_id=(g,), device_id_type=pl.DeviceIdType.MESH,
                ).wait_recv()

            def head_body(h8, carry, d=d, g=g):
                t = d * HG + h8
                slot = lax.rem(t, 2)

                @pl.when(t + 1 < HQ)
                def _():
                    h_next = jnp.where(
                        h8 + 1 < HG,
                        g * HG + h8 + 1,
                        lax.rem(g + 1, N_DEV) * HG)
                    kc, vc = kv_copy(h_next, lax.rem(t + 1, 2))
                    kc.start()
                    vc.start()

                kc, vc = kv_copy(g * HG + h8, slot)
                kc.wait()
                vc.wait()

                qh = q_full[g, :, pl.ds(h8 * DH, DH)]
                kh = k_buf[slot].astype(jnp.bfloat16)
                s = lax.dot_general(
                    qh, kh, (((1,), (1,)), ((), ())),
                    preferred_element_type=jnp.float32)
                s = s * SCALE + bias_ref[...]
                m = jnp.max(s, axis=1)
                w = jnp.exp(s - m[:, None])
                l = jnp.sum(w, axis=1)
                vh = v_buf[slot].astype(jnp.bfloat16)
                o = jnp.dot(w.astype(jnp.bfloat16), vh,
                            preferred_element_type=jnp.float32)
                o_part[g, :, pl.ds(h8 * DH, DH)] = o.astype(jnp.bfloat16)
                ml_part[g, 0, h8, :] = m
                ml_part[g, 1, h8, :] = l
                return carry

            lax.fori_loop(0, HG, head_body, 0)

            if d == 0:
                o_a2a[p, :, :] = o_part[p, :, :]
                ml_a2a[p, :, :, :] = ml_part[p, :, :, :]
            else:
                ro = pltpu.make_async_remote_copy(
                    src_ref=o_part.at[g], dst_ref=o_a2a.at[p],
                    send_sem=o_send.at[g], recv_sem=o_recv.at[p],
                    device_id=(g,), device_id_type=pl.DeviceIdType.MESH)
                rm = pltpu.make_async_remote_copy(
                    src_ref=ml_part.at[g], dst_ref=ml_a2a.at[p],
                    send_sem=ml_send.at[g], recv_sem=ml_recv.at[p],
                    device_id=(g,), device_id_type=pl.DeviceIdType.MESH)
                ro.start()
                rm.start()
                a2a_sends += [ro, rm]

        wait_recvs(o_part, o_a2a, o_send, o_recv)
        wait_recvs(ml_part, ml_a2a, ml_send, ml_recv)
        for r in a2a_sends:
            r.wait_send()
        for r in q_sends:
            r.wait_send()

        mm = ml_a2a[:, 0, :, :]
        ll = ml_a2a[:, 1, :, :]
        mg = jnp.max(mm, axis=0)
        coef = jnp.exp(mm - mg[None, :, :])
        l_tot = jnp.sum(ll * coef, axis=0)
        ctx_cols = []
        for h8 in range(HG):
            oj = o_a2a[:, :, h8 * DH:(h8 + 1) * DH].astype(jnp.float32)
            c = coef[:, h8, :]
            o_tot = jnp.sum(oj * c[:, :, None], axis=0)
            ctx_cols.append(
                (o_tot / l_tot[h8][:, None]).astype(jnp.bfloat16))
        ctx = jnp.concatenate(ctx_cols, axis=1)

        wob = wo_ref[...].astype(jnp.bfloat16)
        out_p = jnp.dot(ctx, wob, preferred_element_type=jnp.float32)
        ar_buf[p, :, :] = out_p.astype(jnp.bfloat16)
        ar_sends = exchange(ar_buf, ar_buf, ar_send, ar_recv)
        wait_recvs(ar_buf, ar_buf, ar_send, ar_recv)
        for r in ar_sends:
            r.wait_send()
        out_ref[0, :, :] = jnp.sum(ar_buf[...].astype(jnp.float32), axis=0)

    f32 = jnp.float32
    bf16 = jnp.bfloat16
    return pl.pallas_call(
        body,
        out_shape=jax.ShapeDtypeStruct((1, SQ, DM), f32),
        in_specs=[
            pl.BlockSpec(memory_space=pltpu.VMEM),
            pl.BlockSpec(memory_space=pltpu.VMEM),
            pl.BlockSpec(memory_space=pl.ANY),
            pl.BlockSpec(memory_space=pl.ANY),
            pl.BlockSpec(memory_space=pltpu.VMEM),
        ],
        out_specs=pl.BlockSpec(memory_space=pltpu.VMEM),
        scratch_shapes=[
            pltpu.VMEM((N_DEV, SQ, DM), bf16),
            pltpu.VMEM((SQ, SKV_LOC), f32),
            pltpu.VMEM((2, SKV_LOC, DH), f32),
            pltpu.VMEM((2, SKV_LOC, DH), f32),
            pltpu.VMEM((N_DEV, SQ, DM), bf16),
            pltpu.VMEM((N_DEV, 2, HG, SQ), f32),
            pltpu.VMEM((N_DEV, SQ, DM), bf16),
            pltpu.VMEM((N_DEV, 2, HG, SQ), f32),
            pltpu.VMEM((N_DEV, SQ, DM), bf16),
            pltpu.SemaphoreType.DMA((2,)),
            pltpu.SemaphoreType.DMA((2,)),
            pltpu.SemaphoreType.DMA((N_DEV,)),
            pltpu.SemaphoreType.DMA((N_DEV,)),
            pltpu.SemaphoreType.DMA((N_DEV,)),
            pltpu.SemaphoreType.DMA((N_DEV,)),
            pltpu.SemaphoreType.DMA((N_DEV,)),
            pltpu.SemaphoreType.DMA((N_DEV,)),
            pltpu.SemaphoreType.DMA((N_DEV,)),
            pltpu.SemaphoreType.DMA((N_DEV,)),
        ],
        compiler_params=pltpu.CompilerParams(collective_id=0),
    )(x, Wq, K_ext, V_ext, Wo)


# device time: 105276 ns/iter; 1.2268x vs baseline; 1.0651x over previous
import jax
import jax.numpy as jnp
from jax import lax
from jax.experimental import pallas as pl
from jax.experimental.pallas import tpu as pltpu

N_DEV = 4
SQ = 256
SKV_LOC = 4096
HQ = 32
HG = 8
DH = 128
DM = 1024
BLK = 64
SCALE = 0.08838834764831843


def kernel(x, Wq, K_ext, V_ext, Wo):
    def body(x_ref, wq_ref, k_hbm, v_hbm, wo_ref, out_ref,
             q_full, bias_ref, k_buf, v_buf, v_aug, o_part, l_part,
             o_a2a, l_a2a, ar_buf,
             k_sem, v_sem, q_send, q_recv, o_send, o_recv,
             ml_send, ml_recv, ar_send, ar_recv):
        p = lax.axis_index("i")

        bar = pltpu.get_barrier_semaphore()
        for d in range(1, N_DEV):
            pl.semaphore_signal(
                bar, inc=1,
                device_id=(lax.rem(p + d, N_DEV),),
                device_id_type=pl.DeviceIdType.MESH,
            )
        pl.semaphore_wait(bar, N_DEV - 1)

        def kv_copy(h, slot):
            kc = pltpu.make_async_copy(
                k_hbm.at[0, :, h, :], k_buf.at[slot], k_sem.at[slot])
            vc = pltpu.make_async_copy(
                v_hbm.at[0, :, h, :], v_buf.at[slot], v_sem.at[slot])
            return kc, vc

        kc0, vc0 = kv_copy(p * HG, 0)
        kc0.start()
        vc0.start()

        xb = x_ref[0].astype(jnp.bfloat16)
        wqb = wq_ref[...].astype(jnp.bfloat16)
        q = jnp.dot(xb, wqb, preferred_element_type=jnp.float32)
        q_full[p, :, :] = (q * SCALE).astype(jnp.bfloat16)

        v_aug[:, DH:] = jnp.zeros((SKV_LOC, DH), jnp.bfloat16)
        v_aug[:, DH:DH + 1] = jnp.ones((SKV_LOC, 1), jnp.bfloat16)

        def exchange(src_buf, dst_buf, send_sems, recv_sems):
            sends = []
            for d in range(1, N_DEV):
                dst = lax.rem(p + d, N_DEV)
                r = pltpu.make_async_remote_copy(
                    src_ref=src_buf.at[p],
                    dst_ref=dst_buf.at[p],
                    send_sem=send_sems.at[dst],
                    recv_sem=recv_sems.at[p],
                    device_id=(dst,),
                    device_id_type=pl.DeviceIdType.MESH,
                )
                r.start()
                sends.append(r)
            return sends

        def wait_recvs(src_buf, dst_buf, send_sems, recv_sems):
            for d in range(1, N_DEV):
                src = lax.rem(p + d, N_DEV)
                pltpu.make_async_remote_copy(
                    src_ref=src_buf.at[src],
                    dst_ref=dst_buf.at[src],
                    send_sem=send_sems.at[src],
                    recv_sem=recv_sems.at[src],
                    device_id=(src,),
                    device_id_type=pl.DeviceIdType.MESH,
                ).wait_recv()

        q_sends = exchange(q_full, q_full, q_send, q_recv)

        ri = lax.broadcasted_iota(jnp.int32, (SQ, SKV_LOC), 0)
        ci = lax.broadcasted_iota(jnp.int32, (SQ, SKV_LOC), 1)
        qb = ri // BLK
        kbg = p * (SKV_LOC // BLK) + ci // BLK
        mask = (qb == kbg) | (kbg == 0) | (lax.rem(qb + kbg, 3) == 0)
        bias_ref[...] = jnp.where(mask, 0.0, -1e9).astype(jnp.float32)

        a2a_sends = []
        for d in range(N_DEV):
            g = lax.rem(p + d, N_DEV)

            if d > 0:
                pltpu.make_async_remote_copy(
                    src_ref=q_full.at[g], dst_ref=q_full.at[g],
                    send_sem=q_send.at[g], recv_sem=q_recv.at[g],
                    device_id=(g,), device_id_type=pl.DeviceIdType.MESH,
                ).wait_recv()

            def head_body(h8, carry, d=d, g=g):
                t = d * HG + h8
                slot = lax.rem(t, 2)

                @pl.when(t + 1 < HQ)
                def _():
                    h_next = jnp.where(
                        h8 + 1 < HG,
                        g * HG + h8 + 1,
                        lax.rem(g + 1, N_DEV) * HG)
                    kc, vc = kv_copy(h_next, lax.rem(t + 1, 2))
                    kc.start()
                    vc.start()

                kc, vc = kv_copy(g * HG + h8, slot)
                kc.wait()
                vc.wait()
                v_aug[:, :DH] = v_buf[slot].astype(jnp.bfloat16)

                qh = q_full[g, :, pl.ds(h8 * DH, DH)]
                HALF = SKV_LOC // 2
                o_acc = None
                for kvh in range(2):
                    kh = k_buf[slot, pl.ds(kvh * HALF, HALF), :]
                    s = lax.dot_general(
                        qh, kh.astype(jnp.bfloat16),
                        (((1,), (1,)), ((), ())),
                        preferred_element_type=jnp.float32)
                    w = jnp.exp(s + bias_ref[:, pl.ds(kvh * HALF, HALF)])
                    ov = jnp.dot(
                        w.astype(jnp.bfloat16),
                        v_aug[pl.ds(kvh * HALF, HALF), :],
                        preferred_element_type=jnp.float32)
                    o_acc = ov if o_acc is None else o_acc + ov
                o_part[g, :, pl.ds(h8 * DH, DH)] = (
                    o_acc[:, :DH].astype(jnp.bfloat16))
                l_part[g, h8, :] = o_acc[:, DH]
                return carry

            lax.fori_loop(0, HG, head_body, 0)

            if d == 0:
                o_a2a[p, :, :] = o_part[p, :, :]
                l_a2a[p, :, :] = l_part[p, :, :]
            else:
                ro = pltpu.make_async_remote_copy(
                    src_ref=o_part.at[g], dst_ref=o_a2a.at[p],
                    send_sem=o_send.at[g], recv_sem=o_recv.at[p],
                    device_id=(g,), device_id_type=pl.DeviceIdType.MESH)
                rm = pltpu.make_async_remote_copy(
                    src_ref=l_part.at[g], dst_ref=l_a2a.at[p],
                    send_sem=ml_send.at[g], recv_sem=ml_recv.at[p],
                    device_id=(g,), device_id_type=pl.DeviceIdType.MESH)
                ro.start()
                rm.start()
                a2a_sends += [ro, rm]

        wait_recvs(o_part, o_a2a, o_send, o_recv)
        wait_recvs(l_part, l_a2a, ml_send, ml_recv)
        for r in a2a_sends:
            r.wait_send()
        for r in q_sends:
            r.wait_send()

        l_tot = jnp.sum(l_a2a[...], axis=0)
        ctx_cols = []
        for h8 in range(HG):
            oj = o_a2a[:, :, h8 * DH:(h8 + 1) * DH].astype(jnp.float32)
            o_tot = jnp.sum(oj, axis=0)
            ctx_cols.append(
                (o_tot / l_tot[h8][:, None]).astype(jnp.bfloat16))
        ctx = jnp.concatenate(ctx_cols, axis=1)

        wob = wo_ref[...].astype(jnp.bfloat16)
        out_p = jnp.dot(ctx, wob, preferred_element_type=jnp.float32)
        ar_buf[p, :, :] = out_p.astype(jnp.bfloat16)
        ar_sends = exchange(ar_buf, ar_buf, ar_send, ar_recv)
        wait_recvs(ar_buf, ar_buf, ar_send, ar_recv)
        for r in ar_sends:
            r.wait_send()
        out_ref[0, :, :] = jnp.sum(ar_buf[...].astype(jnp.float32), axis=0)

    f32 = jnp.float32
    bf16 = jnp.bfloat16
    return pl.pallas_call(
        body,
        out_shape=jax.ShapeDtypeStruct((1, SQ, DM), f32),
        in_specs=[
            pl.BlockSpec(memory_space=pltpu.VMEM),
            pl.BlockSpec(memory_space=pltpu.VMEM),
            pl.BlockSpec(memory_space=pl.ANY),
            pl.BlockSpec(memory_space=pl.ANY),
            pl.BlockSpec(memory_space=pltpu.VMEM),
        ],
        out_specs=pl.BlockSpec(memory_space=pltpu.VMEM),
        scratch_shapes=[
            pltpu.VMEM((N_DEV, SQ, DM), bf16),
            pltpu.VMEM((SQ, SKV_LOC), f32),
            pltpu.VMEM((2, SKV_LOC, DH), f32),
            pltpu.VMEM((2, SKV_LOC, DH), f32),
            pltpu.VMEM((SKV_LOC, 2 * DH), bf16),
            pltpu.VMEM((N_DEV, SQ, DM), bf16),
            pltpu.VMEM((N_DEV, HG, SQ), f32),
            pltpu.VMEM((N_DEV, SQ, DM), bf16),
            pltpu.VMEM((N_DEV, HG, SQ), f32),
            pltpu.VMEM((N_DEV, SQ, DM), bf16),
            pltpu.SemaphoreType.DMA((2,)),
            pltpu.SemaphoreType.DMA((2,)),
            pltpu.SemaphoreType.DMA((N_DEV,)),
            pltpu.SemaphoreType.DMA((N_DEV,)),
            pltpu.SemaphoreType.DMA((N_DEV,)),
            pltpu.SemaphoreType.DMA((N_DEV,)),
            pltpu.SemaphoreType.DMA((N_DEV,)),
            pltpu.SemaphoreType.DMA((N_DEV,)),
            pltpu.SemaphoreType.DMA((N_DEV,)),
            pltpu.SemaphoreType.DMA((N_DEV,)),
        ],
        compiler_params=pltpu.CompilerParams(collective_id=0),
    )(x, Wq, K_ext, V_ext, Wo)


# device time: 89421 ns/iter; 1.4444x vs baseline; 1.1773x over previous
import jax
import jax.numpy as jnp
from jax import lax
from jax.experimental import pallas as pl
from jax.experimental.pallas import tpu as pltpu

N_DEV = 4
SQ = 256
SKV_LOC = 4096
NBLK_LOC = 64
HQ = 32
HG = 8
DH = 128
DM = 1024
BLK = 64
SCALE = 0.08838834764831843
CLS_PAD = 22
CLS_N = CLS_PAD * BLK
NEG = -1e9


def kernel(x, Wq, K_ext, V_ext, Wo):
    def body(x_ref, wq_ref, k_hbm, v_hbm, wo_ref, out_ref,
             q_full, k_buf, v_buf, k_cls, v_cls, o_part, l_part,
             o_a2a, l_a2a, ar_buf,
             k_sem, v_sem, q_send, q_recv, o_send, o_recv,
             ml_send, ml_recv, ar_send, ar_recv):
        p = lax.axis_index("i")

        bar = pltpu.get_barrier_semaphore()
        for d in range(1, N_DEV):
            pl.semaphore_signal(
                bar, inc=1,
                device_id=(lax.rem(p + d, N_DEV),),
                device_id_type=pl.DeviceIdType.MESH,
            )
        pl.semaphore_wait(bar, N_DEV - 1)

        def kv_copy(h, slot):
            kc = pltpu.make_async_copy(
                k_hbm.at[0, :, h, :], k_buf.at[slot], k_sem.at[slot])
            vc = pltpu.make_async_copy(
                v_hbm.at[0, :, h, :], v_buf.at[slot], v_sem.at[slot])
            return kc, vc

        k_cls[1:3, (CLS_PAD - 1) * BLK:, :] = jnp.zeros(
            (2, BLK, DH), jnp.bfloat16)
        v_cls[1:3, (CLS_PAD - 1) * BLK:, :] = jnp.zeros(
            (2, BLK, DH), jnp.bfloat16)

        kc0, vc0 = kv_copy(p * HG, 0)
        kc0.start()
        vc0.start()

        xb = x_ref[0].astype(jnp.bfloat16)
        wqb = wq_ref[...].astype(jnp.bfloat16)
        q = jnp.dot(xb, wqb, preferred_element_type=jnp.float32)
        q_full[p, :, :] = (q * SCALE).astype(jnp.bfloat16)

        def exchange(src_buf, dst_buf, send_sems, recv_sems):
            sends = []
            for d in range(1, N_DEV):
                dst = lax.rem(p + d, N_DEV)
                r = pltpu.make_async_remote_copy(
                    src_ref=src_buf.at[p],
                    dst_ref=dst_buf.at[p],
                    send_sem=send_sems.at[dst],
                    recv_sem=recv_sems.at[p],
                    device_id=(dst,),
                    device_id_type=pl.DeviceIdType.MESH,
                )
                r.start()
                sends.append(r)
            return sends

        def wait_recvs(src_buf, dst_buf, send_sems, recv_sems):
            for d in range(1, N_DEV):
                src = lax.rem(p + d, N_DEV)
                pltpu.make_async_remote_copy(
                    src_ref=src_buf.at[src],
                    dst_ref=dst_buf.at[src],
                    send_sem=send_sems.at[src],
                    recv_sem=recv_sems.at[src],
                    device_id=(src,),
                    device_id_type=pl.DeviceIdType.MESH,
                ).wait_recv()

        q_sends = exchange(q_full, q_full, q_send, q_recv)

        ccol = lax.broadcasted_iota(jnp.int32, (1, CLS_N), 1)
        tail22 = jnp.zeros((1, CLS_N), jnp.float32)
        tail21 = jnp.where(ccol < (CLS_PAD - 1) * BLK, 0.0, NEG)
        xrow = lax.broadcasted_iota(jnp.int32, (2 * BLK, 4 * BLK), 0)
        xcol = lax.broadcasted_iota(jnp.int32, (2 * BLK, 4 * BLK), 1) // BLK
        is_p0 = p == 0
        x_allowed = ((xcol == 0)
                     | ((xrow < BLK) & (xcol == 1))
                     | ((xrow >= BLK) & (xcol == 2)))
        xbias = jnp.where(is_p0 & x_allowed, 0.0, NEG)

        a2a_sends = []
        for d in range(N_DEV):
            g = lax.rem(p + d, N_DEV)

            if d > 0:
                pltpu.make_async_remote_copy(
                    src_ref=q_full.at[g], dst_ref=q_full.at[g],
                    send_sem=q_send.at[g], recv_sem=q_recv.at[g],
                    device_id=(g,), device_id_type=pl.DeviceIdType.MESH,
                ).wait_recv()

            def head_body(h8, carry, d=d, g=g):
                t = d * HG + h8
                slot = lax.rem(t, 2)

                @pl.when(t + 1 < HQ)
                def _():
                    h_next = jnp.where(
                        h8 + 1 < HG,
                        g * HG + h8 + 1,
                        lax.rem(g + 1, N_DEV) * HG)
                    kc, vc = kv_copy(h_next, lax.rem(t + 1, 2))
                    kc.start()
                    vc.start()

                kc, vc = kv_copy(g * HG + h8, slot)
                kc.wait()
                vc.wait()

                for r in range(3):
                    nb = CLS_PAD if r == 0 else CLS_PAD - 1
                    kcat = jnp.concatenate(
                        [k_buf[slot, (r + 3 * k) * BLK:(r + 3 * k + 1) * BLK, :]
                         for k in range(nb)], axis=0)
                    k_cls[r, :nb * BLK, :] = kcat.astype(jnp.bfloat16)
                    vcat = jnp.concatenate(
                        [v_buf[slot, (r + 3 * k) * BLK:(r + 3 * k + 1) * BLK, :]
                         for k in range(nb)], axis=0)
                    v_cls[r, :nb * BLK, :] = vcat.astype(jnp.bfloat16)

                k_x = k_buf[slot, :4 * BLK, :].astype(jnp.bfloat16)
                v_x = v_buf[slot, :4 * BLK, :].astype(jnp.bfloat16)

                dsh = pl.ds(h8 * DH, DH)

                def chain(qh, r_c):
                    s = lax.dot_general(
                        qh, k_cls[r_c], (((1,), (1,)), ((), ())),
                        preferred_element_type=jnp.float32)
                    w = jnp.exp(s + jnp.where(r_c == 0, tail22, tail21))
                    return (jnp.dot(w.astype(jnp.bfloat16), v_cls[r_c],
                                    preferred_element_type=jnp.float32),
                            jnp.sum(w, axis=1))

                q03 = jnp.concatenate(
                    [q_full[g, 0:BLK, dsh],
                     q_full[g, 3 * BLK:4 * BLK, dsh]], axis=0)
                o03, l03 = chain(q03, lax.rem(3 - lax.rem(p, 3), 3))
                q1 = q_full[g, pl.ds(BLK, BLK), dsh]
                o1, l1 = chain(q1, lax.rem(3 - lax.rem(1 + p, 3), 3))
                q2 = q_full[g, pl.ds(2 * BLK, BLK), dsh]
                o2, l2 = chain(q2, lax.rem(3 - lax.rem(2 + p, 3), 3))
                q12 = q_full[g, pl.ds(BLK, 2 * BLK), dsh]
                s_x = lax.dot_general(
                    q12, k_x, (((1,), (1,)), ((), ())),
                    preferred_element_type=jnp.float32)
                w_x = jnp.exp(s_x + xbias)
                l_x = jnp.sum(w_x, axis=1)
                o_x = jnp.dot(w_x.astype(jnp.bfloat16), v_x,
                              preferred_element_type=jnp.float32)

                o_part[g, pl.ds(0, BLK), dsh] = (
                    o03[:BLK].astype(jnp.bfloat16))
                o_part[g, pl.ds(BLK, BLK), dsh] = (
                    (o1 + o_x[:BLK]).astype(jnp.bfloat16))
                o_part[g, pl.ds(2 * BLK, BLK), dsh] = (
                    (o2 + o_x[BLK:]).astype(jnp.bfloat16))
                o_part[g, pl.ds(3 * BLK, BLK), dsh] = (
                    o03[BLK:].astype(jnp.bfloat16))
                l_part[g, h8, :] = jnp.concatenate(
                    [l03[:BLK], l1 + l_x[:BLK], l2 + l_x[BLK:], l03[BLK:]])
                return carry

            lax.fori_loop(0, HG, head_body, 0)

            if d == 0:
                o_a2a[p, :, :] = o_part[p, :, :]
                l_a2a[p, :, :] = l_part[p, :, :]
            else:
                ro = pltpu.make_async_remote_copy(
                    src_ref=o_part.at[g], dst_ref=o_a2a.at[p],
                    send_sem=o_send.at[g], recv_sem=o_recv.at[p],
                    device_id=(g,), device_id_type=pl.DeviceIdType.MESH)
                rm = pltpu.make_async_remote_copy(
                    src_ref=l_part.at[g], dst_ref=l_a2a.at[p],
                    send_sem=ml_send.at[g], recv_sem=ml_recv.at[p],
                    device_id=(g,), device_id_type=pl.DeviceIdType.MESH)
                ro.start()
                rm.start()
                a2a_sends += [ro, rm]

        wait_recvs(o_part, o_a2a, o_send, o_recv)
        wait_recvs(l_part, l_a2a, ml_send, ml_recv)
        for r in a2a_sends:
            r.wait_send()
        for r in q_sends:
            r.wait_send()

        l_tot = jnp.sum(l_a2a[...], axis=0)
        ctx_cols = []
        for h8 in range(HG):
            oj = o_a2a[:, :, h8 * DH:(h8 + 1) * DH].astype(jnp.float32)
            o_tot = jnp.sum(oj, axis=0)
            ctx_cols.append(
                (o_tot / l_tot[h8][:, None]).astype(jnp.bfloat16))
        ctx = jnp.concatenate(ctx_cols, axis=1)

        wob = wo_ref[...].astype(jnp.bfloat16)
        out_p = jnp.dot(ctx, wob, preferred_element_type=jnp.float32)
        ar_buf[p, :, :] = out_p.astype(jnp.bfloat16)
        ar_sends = exchange(ar_buf, ar_buf, ar_send, ar_recv)
        wait_recvs(ar_buf, ar_buf, ar_send, ar_recv)
        for r in ar_sends:
            r.wait_send()
        out_ref[0, :, :] = jnp.sum(ar_buf[...].astype(jnp.float32), axis=0)

    f32 = jnp.float32
    bf16 = jnp.bfloat16
    return pl.pallas_call(
        body,
        out_shape=jax.ShapeDtypeStruct((1, SQ, DM), f32),
        in_specs=[
            pl.BlockSpec(memory_space=pltpu.VMEM),
            pl.BlockSpec(memory_space=pltpu.VMEM),
            pl.BlockSpec(memory_space=pl.ANY),
            pl.BlockSpec(memory_space=pl.ANY),
            pl.BlockSpec(memory_space=pltpu.VMEM),
        ],
        out_specs=pl.BlockSpec(memory_space=pltpu.VMEM),
        scratch_shapes=[
            pltpu.VMEM((N_DEV, SQ, DM), bf16),
            pltpu.VMEM((2, SKV_LOC, DH), f32),
            pltpu.VMEM((2, SKV_LOC, DH), f32),
            pltpu.VMEM((3, CLS_N, DH), bf16),
            pltpu.VMEM((3, CLS_N, DH), bf16),
            pltpu.VMEM((N_DEV, SQ, DM), bf16),
            pltpu.VMEM((N_DEV, HG, SQ), f32),
            pltpu.VMEM((N_DEV, SQ, DM), bf16),
            pltpu.VMEM((N_DEV, HG, SQ), f32),
            pltpu.VMEM((N_DEV, SQ, DM), bf16),
            pltpu.SemaphoreType.DMA((2,)),
            pltpu.SemaphoreType.DMA((2,)),
            pltpu.SemaphoreType.DMA((N_DEV,)),
            pltpu.SemaphoreType.DMA((N_DEV,)),
            pltpu.SemaphoreType.DMA((N_DEV,)),
            pltpu.SemaphoreType.DMA((N_DEV,)),
            pltpu.SemaphoreType.DMA((N_DEV,)),
            pltpu.SemaphoreType.DMA((N_DEV,)),
            pltpu.SemaphoreType.DMA((N_DEV,)),
            pltpu.SemaphoreType.DMA((N_DEV,)),
        ],
        compiler_params=pltpu.CompilerParams(
            collective_id=0, vmem_limit_bytes=60 * 1024 * 1024),
    )(x, Wq, K_ext, V_ext, Wo)
